# initial kernel scaffold (unmeasured)
import jax
import jax.numpy as jnp
from jax import lax
from jax.experimental import pallas as pl
from jax.experimental.pallas import tpu as pltpu

N_DEV = 4


def kernel(x, w_mat):
    partial = jnp.dot(x, w_mat, preferred_element_type=jnp.float32)
    partial = partial.astype(jnp.bfloat16)
    M, N = partial.shape
    MC = M // N_DEV

    def body(p_hbm, out_hbm, send_buf, recv_bufs, local_buf, f32_buf,
             send_sems, recv_sems, local_sem, out_sem, credit_sem):
        my = lax.axis_index("i")
        left = lax.rem(my + N_DEV - 1, N_DEV)
        right = lax.rem(my + 1, N_DEV)

        def mod4(v):
            return lax.rem(v + 2 * N_DEV, N_DEV)

        pre = pltpu.make_async_copy(
            p_hbm.at[pl.ds(my * MC, MC), :], send_buf, local_sem)
        pre.start()

        barrier = pltpu.get_barrier_semaphore()
        pl.semaphore_signal(barrier, inc=1, device_id=(left,),
                            device_id_type=pl.DeviceIdType.MESH)
        pl.semaphore_signal(barrier, inc=1, device_id=(right,),
                            device_id_type=pl.DeviceIdType.MESH)
        pl.semaphore_wait(barrier, 2)
        pre.wait()

        out_copy = None
        n_hops = 2 * (N_DEV - 1)
        for h in range(n_hops):
            slot = h % 2
            if h >= 2:
                pl.semaphore_wait(credit_sem, 1)
            rdma = pltpu.make_async_remote_copy(
                src_ref=send_buf,
                dst_ref=recv_bufs.at[slot],
                send_sem=send_sems.at[slot],
                recv_sem=recv_sems.at[slot],
                device_id=(right,),
                device_id_type=pl.DeviceIdType.MESH,
            )
            rdma.start()
            lcopy = None
            if h < N_DEV - 1:
                c_next = mod4(my - h - 1)
                lcopy = pltpu.make_async_copy(
                    p_hbm.at[pl.ds(c_next * MC, MC), :], local_buf,
                    local_sem)
                lcopy.start()
            rdma.wait()
            if lcopy is not None:
                lcopy.wait()

            if h < N_DEV - 2:
                acc = (recv_bufs[slot].astype(jnp.float32)
                       + local_buf[...].astype(jnp.float32))
                send_buf[...] = acc.astype(jnp.bfloat16)
            elif h == N_DEV - 2:
                acc = (recv_bufs[slot].astype(jnp.float32)
                       + local_buf[...].astype(jnp.float32))
                y = acc * jax.nn.sigmoid(acc)
                f32_buf[...] = y
                send_buf[...] = y.astype(jnp.bfloat16)
                out_copy = pltpu.make_async_copy(
                    f32_buf, out_hbm.at[pl.ds(mod4(my + 1) * MC, MC), :],
                    out_sem)
                out_copy.start()
            else:
                a = h - (N_DEV - 1)
                c = mod4(my - a)
                out_copy.wait()
                f32_buf[...] = recv_bufs[slot].astype(jnp.float32)
                out_copy = pltpu.make_async_copy(
                    f32_buf, out_hbm.at[pl.ds(c * MC, MC), :], out_sem)
                out_copy.start()
                if h < n_hops - 1:
                    send_buf[...] = recv_bufs[slot]
            if h <= 3:
                pl.semaphore_signal(credit_sem, inc=1, device_id=(left,),
                                    device_id_type=pl.DeviceIdType.MESH)
        out_copy.wait()

    return pl.pallas_call(
        body,
        out_shape=jax.ShapeDtypeStruct((M, N), jnp.float32),
        in_specs=[pl.BlockSpec(memory_space=pltpu.ANY)],
        out_specs=pl.BlockSpec(memory_space=pltpu.ANY),
        scratch_shapes=[
            pltpu.VMEM((MC, N), jnp.bfloat16),
            pltpu.VMEM((2, MC, N), jnp.bfloat16),
            pltpu.VMEM((MC, N), jnp.bfloat16),
            pltpu.VMEM((MC, N), jnp.float32),
            pltpu.SemaphoreType.DMA((2,)),
            pltpu.SemaphoreType.DMA((2,)),
            pltpu.SemaphoreType.DMA,
            pltpu.SemaphoreType.DMA,
            pltpu.SemaphoreType.REGULAR,
        ],
        compiler_params=pltpu.CompilerParams(collective_id=0),
    )(partial)


# baseline (device time: 1441452 ns/iter reference)
import jax
import jax.numpy as jnp
from jax import lax
from jax.experimental import pallas as pl
from jax.experimental.pallas import tpu as pltpu

N_DEV = 4
G = 4


def kernel(x, w_mat):
    partial = jnp.dot(x, w_mat, preferred_element_type=jnp.float32)
    partial = partial.astype(jnp.bfloat16)
    M, N = partial.shape
    MC = M // N_DEV
    NW = N // G

    n_hops = 2 * (N_DEV - 1)
    total_hops = G * n_hops

    def body(p_hbm, out_hbm, send_buf, recv_bufs, local_buf, f32_buf,
             send_sems, recv_sems, local_sem, out_sem, credit_sem):
        my = lax.axis_index("i")
        left = lax.rem(my + N_DEV - 1, N_DEV)
        right = lax.rem(my + 1, N_DEV)

        def mod4(v):
            return lax.rem(v + 2 * N_DEV, N_DEV)

        pre = pltpu.make_async_copy(
            p_hbm.at[pl.ds(my * MC, MC), pl.ds(0, NW)], send_buf,
            local_sem)
        pre.start()

        barrier = pltpu.get_barrier_semaphore()
        pl.semaphore_signal(barrier, inc=1, device_id=(left,),
                            device_id_type=pl.DeviceIdType.MESH)
        pl.semaphore_signal(barrier, inc=1, device_id=(right,),
                            device_id_type=pl.DeviceIdType.MESH)
        pl.semaphore_wait(barrier, 2)
        pre.wait()

        out_copy = None
        for g in range(G):
            cols = pl.ds(g * NW, NW)
            for h in range(n_hops):
                gh = g * n_hops + h
                slot = gh % 2
                if gh >= 2:
                    pl.semaphore_wait(credit_sem, 1)
                rdma = pltpu.make_async_remote_copy(
                    src_ref=send_buf,
                    dst_ref=recv_bufs.at[slot],
                    send_sem=send_sems.at[slot],
                    recv_sem=recv_sems.at[slot],
                    device_id=(right,),
                    device_id_type=pl.DeviceIdType.MESH,
                )
                rdma.start()
                lcopy = None
                if h < N_DEV - 1:
                    c_next = mod4(my - h - 1)
                    lcopy = pltpu.make_async_copy(
                        p_hbm.at[pl.ds(c_next * MC, MC), cols],
                        local_buf, local_sem)
                    lcopy.start()
                rdma.wait()
                if lcopy is not None:
                    lcopy.wait()

                if h < N_DEV - 2:
                    acc = (recv_bufs[slot].astype(jnp.float32)
                           + local_buf[...].astype(jnp.float32))
                    send_buf[...] = acc.astype(jnp.bfloat16)
                elif h == N_DEV - 2:
                    acc = (recv_bufs[slot].astype(jnp.float32)
                           + local_buf[...].astype(jnp.float32))
                    y = acc * jax.nn.sigmoid(acc)
                    if out_copy is not None:
                        out_copy.wait()
                    f32_buf[...] = y
                    send_buf[...] = y.astype(jnp.bfloat16)
                    out_copy = pltpu.make_async_copy(
                        f32_buf,
                        out_hbm.at[pl.ds(mod4(my + 1) * MC, MC), cols],
                        out_sem)
                    out_copy.start()
                else:
                    a = h - (N_DEV - 1)
                    c = mod4(my - a)
                    out_copy.wait()
                    f32_buf[...] = recv_bufs[slot].astype(jnp.float32)
                    out_copy = pltpu.make_async_copy(
                        f32_buf, out_hbm.at[pl.ds(c * MC, MC), cols],
                        out_sem)
                    out_copy.start()
                    if h < n_hops - 1:
                        send_buf[...] = recv_bufs[slot]
                if gh + 2 <= total_hops - 1:
                    pl.semaphore_signal(
                        credit_sem, inc=1, device_id=(left,),
                        device_id_type=pl.DeviceIdType.MESH)
            if g + 1 < G:
                pre = pltpu.make_async_copy(
                    p_hbm.at[pl.ds(my * MC, MC), pl.ds((g + 1) * NW, NW)],
                    send_buf, local_sem)
                pre.start()
                pre.wait()
        out_copy.wait()

    return pl.pallas_call(
        body,
        out_shape=jax.ShapeDtypeStruct((M, N), jnp.float32),
        in_specs=[pl.BlockSpec(memory_space=pl.ANY)],
        out_specs=pl.BlockSpec(memory_space=pl.ANY),
        scratch_shapes=[
            pltpu.VMEM((MC, NW), jnp.bfloat16),
            pltpu.VMEM((2, MC, NW), jnp.bfloat16),
            pltpu.VMEM((MC, NW), jnp.bfloat16),
            pltpu.VMEM((MC, NW), jnp.float32),
            pltpu.SemaphoreType.DMA((2,)),
            pltpu.SemaphoreType.DMA((2,)),
            pltpu.SemaphoreType.DMA,
            pltpu.SemaphoreType.DMA,
            pltpu.SemaphoreType.REGULAR,
        ],
        compiler_params=pltpu.CompilerParams(
            collective_id=0, vmem_limit_bytes=64 * 1024 * 1024),
    )(partial)


# device time: 885563 ns/iter; 1.6277x vs baseline; 1.6277x over previous
import jax
import jax.numpy as jnp
from jax import lax
from jax.experimental import pallas as pl
from jax.experimental.pallas import tpu as pltpu

N_DEV = 4
G = 8
R = G // 2
N_HOPS = 2 * (N_DEV - 1)


def kernel(x, w_mat):
    partial = jnp.dot(x, w_mat, preferred_element_type=jnp.float32)
    partial = partial.astype(jnp.bfloat16)
    M, N = partial.shape
    MC = M // N_DEV
    NW = N // G

    def body(p_hbm, out_hbm,
             sbuf0, sbuf1, rbufs0, rbufs1, lbuf0, lbuf1, fbuf0, fbuf1,
             ssem0, ssem1, rsem0, rsem1, lsem0, lsem1, osem0, osem1,
             cred0, cred1):
        my = lax.axis_index("i")
        left = lax.rem(my + N_DEV - 1, N_DEV)
        right = lax.rem(my + 1, N_DEV)

        def mod4(v):
            return lax.rem(v + 2 * N_DEV, N_DEV)

        dirs = [
            dict(sbuf=sbuf0, rbufs=rbufs0, lbuf=lbuf0, fbuf=fbuf0,
                 ssem=ssem0, rsem=rsem0, lsem=lsem0, osem=osem0,
                 cred=cred0, tgt=right, crd_tgt=left, sgn=1, g0=0,
                 out_copy=None, pre=None, rdma=None, lcopy=None),
            dict(sbuf=sbuf1, rbufs=rbufs1, lbuf=lbuf1, fbuf=fbuf1,
                 ssem=ssem1, rsem=rsem1, lsem=lsem1, osem=osem1,
                 cred=cred1, tgt=left, crd_tgt=right, sgn=-1, g0=R,
                 out_copy=None, pre=None, rdma=None, lcopy=None),
        ]

        def signal_credit(D):
            pl.semaphore_signal(D["cred"], inc=1,
                                device_id=(D["crd_tgt"],),
                                device_id_type=pl.DeviceIdType.MESH)

        for D in dirs:
            D["pre"] = pltpu.make_async_copy(
                p_hbm.at[pl.ds(my * MC, MC), pl.ds(D["g0"] * NW, NW)],
                D["sbuf"], D["lsem"])
            D["pre"].start()

        barrier = pltpu.get_barrier_semaphore()
        pl.semaphore_signal(barrier, inc=1, device_id=(left,),
                            device_id_type=pl.DeviceIdType.MESH)
        pl.semaphore_signal(barrier, inc=1, device_id=(right,),
                            device_id_type=pl.DeviceIdType.MESH)
        pl.semaphore_wait(barrier, 2)

        for r in range(R):
            for h in range(N_HOPS):
                gh = r * N_HOPS + h
                slot = gh % 2
                for D in dirs:
                    if gh >= 2:
                        pl.semaphore_wait(D["cred"], 1)
                    if h == 0:
                        D["pre"].wait()
                    src = D["sbuf"] if h <= 3 else D["rbufs"].at[(gh - 1) % 2]
                    D["rdma"] = pltpu.make_async_remote_copy(
                        src_ref=src,
                        dst_ref=D["rbufs"].at[slot],
                        send_sem=D["ssem"].at[slot],
                        recv_sem=D["rsem"].at[slot],
                        device_id=(D["tgt"],),
                        device_id_type=pl.DeviceIdType.MESH,
                    )
                    D["rdma"].start()
                    if h < N_DEV - 1:
                        c_next = mod4(my - D["sgn"] * (h + 1))
                        D["lcopy"] = pltpu.make_async_copy(
                            p_hbm.at[pl.ds(c_next * MC, MC),
                                     pl.ds((D["g0"] + r) * NW, NW)],
                            D["lbuf"], D["lsem"])
                        D["lcopy"].start()
                for D in dirs:
                    cols = pl.ds((D["g0"] + r) * NW, NW)
                    D["rdma"].wait()
                    if h < N_DEV - 1:
                        D["lcopy"].wait()
                    if h < N_DEV - 2:
                        acc = (D["rbufs"][slot].astype(jnp.float32)
                               + D["lbuf"][...].astype(jnp.float32))
                        D["sbuf"][...] = acc.astype(jnp.bfloat16)
                        signal_credit(D)
                    elif h == N_DEV - 2:
                        acc = (D["rbufs"][slot].astype(jnp.float32)
                               + D["lbuf"][...].astype(jnp.float32))
                        y = acc * jax.nn.sigmoid(acc)
                        if D["out_copy"] is not None:
                            D["out_copy"].wait()
                        D["fbuf"][...] = y
                        D["sbuf"][...] = y.astype(jnp.bfloat16)
                        own = mod4(my + D["sgn"])
                        D["out_copy"] = pltpu.make_async_copy(
                            D["fbuf"], out_hbm.at[pl.ds(own * MC, MC), cols],
                            D["osem"])
                        D["out_copy"].start()
                        signal_credit(D)
                    else:
                        a = h - (N_DEV - 1)
                        c = mod4(my - D["sgn"] * a)
                        D["out_copy"].wait()
                        D["fbuf"][...] = D["rbufs"][slot].astype(jnp.float32)
                        D["out_copy"] = pltpu.make_async_copy(
                            D["fbuf"], out_hbm.at[pl.ds(c * MC, MC), cols],
                            D["osem"])
                        D["out_copy"].start()
                        if h == 3 and r + 1 < R:
                            D["pre"] = pltpu.make_async_copy(
                                p_hbm.at[pl.ds(my * MC, MC),
                                         pl.ds((D["g0"] + r + 1) * NW, NW)],
                                D["sbuf"], D["lsem"])
                            D["pre"].start()
                        if h == 4:
                            signal_credit(D)
                        if h == 5 and r + 1 < R:
                            signal_credit(D)
                            signal_credit(D)
        for D in dirs:
            D["out_copy"].wait()

    return pl.pallas_call(
        body,
        out_shape=jax.ShapeDtypeStruct((M, N), jnp.float32),
        in_specs=[pl.BlockSpec(memory_space=pl.ANY)],
        out_specs=pl.BlockSpec(memory_space=pl.ANY),
        scratch_shapes=[
            pltpu.VMEM((MC, NW), jnp.bfloat16),
            pltpu.VMEM((MC, NW), jnp.bfloat16),
            pltpu.VMEM((2, MC, NW), jnp.bfloat16),
            pltpu.VMEM((2, MC, NW), jnp.bfloat16),
            pltpu.VMEM((MC, NW), jnp.bfloat16),
            pltpu.VMEM((MC, NW), jnp.bfloat16),
            pltpu.VMEM((MC, NW), jnp.float32),
            pltpu.VMEM((MC, NW), jnp.float32),
            pltpu.SemaphoreType.DMA((2,)),
            pltpu.SemaphoreType.DMA((2,)),
            pltpu.SemaphoreType.DMA((2,)),
            pltpu.SemaphoreType.DMA((2,)),
            pltpu.SemaphoreType.DMA,
            pltpu.SemaphoreType.DMA,
            pltpu.SemaphoreType.DMA,
            pltpu.SemaphoreType.DMA,
            pltpu.SemaphoreType.REGULAR,
            pltpu.SemaphoreType.REGULAR,
        ],
        compiler_params=pltpu.CompilerParams(
            collective_id=0, vmem_limit_bytes=64 * 1024 * 1024),
    )(partial)


# device time: 885533 ns/iter; 1.6278x vs baseline; 1.0000x over previous
import jax
import jax.numpy as jnp
from jax import lax
from jax.experimental import pallas as pl
from jax.experimental.pallas import tpu as pltpu

N_DEV = 4
G = 8
R = G // 2
N_HOPS = 2 * (N_DEV - 1)


def kernel(x, w_mat):
    partial = jnp.dot(x, w_mat, preferred_element_type=jnp.bfloat16)
    M, N = partial.shape
    MC = M // N_DEV
    NW = N // G

    def body(p_hbm, out_hbm,
             sbuf0, sbuf1, rbufs0, rbufs1, lbuf0, lbuf1, fbuf0, fbuf1,
             ssem0, ssem1, rsem0, rsem1, lsem0, lsem1, osem0, osem1,
             cred0, cred1):
        my = lax.axis_index("i")
        left = lax.rem(my + N_DEV - 1, N_DEV)
        right = lax.rem(my + 1, N_DEV)

        def mod4(v):
            return lax.rem(v + 2 * N_DEV, N_DEV)

        dirs = [
            dict(sbuf=sbuf0, rbufs=rbufs0, lbuf=lbuf0, fbuf=fbuf0,
                 ssem=ssem0, rsem=rsem0, lsem=lsem0, osem=osem0,
                 cred=cred0, tgt=right, crd_tgt=left, sgn=1, g0=0,
                 out_copy=None, pre=None, rdma=None, lcopy=None),
            dict(sbuf=sbuf1, rbufs=rbufs1, lbuf=lbuf1, fbuf=fbuf1,
                 ssem=ssem1, rsem=rsem1, lsem=lsem1, osem=osem1,
                 cred=cred1, tgt=left, crd_tgt=right, sgn=-1, g0=R,
                 out_copy=None, pre=None, rdma=None, lcopy=None),
        ]

        def signal_credit(D):
            pl.semaphore_signal(D["cred"], inc=1,
                                device_id=(D["crd_tgt"],),
                                device_id_type=pl.DeviceIdType.MESH)

        for D in dirs:
            D["pre"] = pltpu.make_async_copy(
                p_hbm.at[pl.ds(my * MC, MC), pl.ds(D["g0"] * NW, NW)],
                D["sbuf"], D["lsem"])
            D["pre"].start()

        barrier = pltpu.get_barrier_semaphore()
        pl.semaphore_signal(barrier, inc=1, device_id=(left,),
                            device_id_type=pl.DeviceIdType.MESH)
        pl.semaphore_signal(barrier, inc=1, device_id=(right,),
                            device_id_type=pl.DeviceIdType.MESH)
        pl.semaphore_wait(barrier, 2)

        for r in range(R):
            for h in range(N_HOPS):
                gh = r * N_HOPS + h
                slot = gh % 2
                for D in dirs:
                    if gh >= 2:
                        pl.semaphore_wait(D["cred"], 1)
                    if h == 0:
                        D["pre"].wait()
                    src = D["sbuf"] if h <= 3 else D["rbufs"].at[(gh - 1) % 2]
                    D["rdma"] = pltpu.make_async_remote_copy(
                        src_ref=src,
                        dst_ref=D["rbufs"].at[slot],
                        send_sem=D["ssem"].at[slot],
                        recv_sem=D["rsem"].at[slot],
                        device_id=(D["tgt"],),
                        device_id_type=pl.DeviceIdType.MESH,
                    )
                    D["rdma"].start()
                    if h < N_DEV - 1:
                        c_next = mod4(my - D["sgn"] * (h + 1))
                        D["lcopy"] = pltpu.make_async_copy(
                            p_hbm.at[pl.ds(c_next * MC, MC),
                                     pl.ds((D["g0"] + r) * NW, NW)],
                            D["lbuf"], D["lsem"])
                        D["lcopy"].start()
                for D in dirs:
                    cols = pl.ds((D["g0"] + r) * NW, NW)
                    D["rdma"].wait()
                    if h < N_DEV - 1:
                        D["lcopy"].wait()
                    if h < N_DEV - 2:
                        acc = (D["rbufs"][slot].astype(jnp.float32)
                               + D["lbuf"][...].astype(jnp.float32))
                        D["sbuf"][...] = acc.astype(jnp.bfloat16)
                        signal_credit(D)
                    elif h == N_DEV - 2:
                        acc = (D["rbufs"][slot].astype(jnp.float32)
                               + D["lbuf"][...].astype(jnp.float32))
                        y = acc * jax.nn.sigmoid(acc)
                        if D["out_copy"] is not None:
                            D["out_copy"].wait()
                        D["fbuf"][...] = y
                        D["sbuf"][...] = y.astype(jnp.bfloat16)
                        own = mod4(my + D["sgn"])
                        D["out_copy"] = pltpu.make_async_copy(
                            D["fbuf"], out_hbm.at[pl.ds(own * MC, MC), cols],
                            D["osem"])
                        D["out_copy"].start()
                        signal_credit(D)
                    else:
                        a = h - (N_DEV - 1)
                        c = mod4(my - D["sgn"] * a)
                        D["out_copy"].wait()
                        D["fbuf"][...] = D["rbufs"][slot].astype(jnp.float32)
                        D["out_copy"] = pltpu.make_async_copy(
                            D["fbuf"], out_hbm.at[pl.ds(c * MC, MC), cols],
                            D["osem"])
                        D["out_copy"].start()
                        if h == 3 and r + 1 < R:
                            D["pre"] = pltpu.make_async_copy(
                                p_hbm.at[pl.ds(my * MC, MC),
                                         pl.ds((D["g0"] + r + 1) * NW, NW)],
                                D["sbuf"], D["lsem"])
                            D["pre"].start()
                        if h == 4:
                            signal_credit(D)
                        if h == 5 and r + 1 < R:
                            signal_credit(D)
                            signal_credit(D)
        for D in dirs:
            D["out_copy"].wait()

    return pl.pallas_call(
        body,
        out_shape=jax.ShapeDtypeStruct((M, N), jnp.float32),
        in_specs=[pl.BlockSpec(memory_space=pl.ANY)],
        out_specs=pl.BlockSpec(memory_space=pl.ANY),
        scratch_shapes=[
            pltpu.VMEM((MC, NW), jnp.bfloat16),
            pltpu.VMEM((MC, NW), jnp.bfloat16),
            pltpu.VMEM((2, MC, NW), jnp.bfloat16),
            pltpu.VMEM((2, MC, NW), jnp.bfloat16),
            pltpu.VMEM((MC, NW), jnp.bfloat16),
            pltpu.VMEM((MC, NW), jnp.bfloat16),
            pltpu.VMEM((MC, NW), jnp.float32),
            pltpu.VMEM((MC, NW), jnp.float32),
            pltpu.SemaphoreType.DMA((2,)),
            pltpu.SemaphoreType.DMA((2,)),
            pltpu.SemaphoreType.DMA((2,)),
            pltpu.SemaphoreType.DMA((2,)),
            pltpu.SemaphoreType.DMA,
            pltpu.SemaphoreType.DMA,
            pltpu.SemaphoreType.DMA,
            pltpu.SemaphoreType.DMA,
            pltpu.SemaphoreType.REGULAR,
            pltpu.SemaphoreType.REGULAR,
        ],
        compiler_params=pltpu.CompilerParams(
            collective_id=0, vmem_limit_bytes=64 * 1024 * 1024),
    )(partial)


# device time: 817646 ns/iter; 1.7629x vs baseline; 1.0830x over previous
import jax
import jax.numpy as jnp
from jax import lax
from jax.experimental import pallas as pl
from jax.experimental.pallas import tpu as pltpu

N_DEV = 4
G = 8
R_LANE = 2
N_HOPS = 2 * (N_DEV - 1)
LS_TOT = R_LANE * N_HOPS


def kernel(x, w_mat):
    partial = jnp.dot(x, w_mat, preferred_element_type=jnp.bfloat16)
    M, N = partial.shape
    MC = M // N_DEV
    NW = N // G

    def body(p_hbm, out_hbm, *s):
        (sb0, sb1, sb2, sb3,
         rb0, rb1, rb2, rb3,
         lb0, lb1, lb2, lb3,
         fb0, fb1,
         ss0, ss1, ss2, ss3,
         rs0, rs1, rs2, rs3,
         ls0, ls1, ls2, ls3,
         os0, os1,
         cr0, cr1, cr2, cr3) = s
        my = lax.axis_index("i")
        left = lax.rem(my + N_DEV - 1, N_DEV)
        right = lax.rem(my + 1, N_DEV)

        def mod4(v):
            return lax.rem(v + 2 * N_DEV, N_DEV)

        dirstate = [
            dict(fbuf=fb0, osem=os0, out_copy=None),
            dict(fbuf=fb1, osem=os1, out_copy=None),
        ]
        lanes = [
            dict(d=0, ph=0, sbuf=sb0, rbufs=rb0, lbuf=lb0, ssem=ss0,
                 rsem=rs0, lsem=ls0, cred=cr0, tgt=right, crd_tgt=left,
                 sgn=1, pre=None, rdma=None, lcopy=None),
            dict(d=0, ph=1, sbuf=sb1, rbufs=rb1, lbuf=lb1, ssem=ss1,
                 rsem=rs1, lsem=ls1, cred=cr1, tgt=right, crd_tgt=left,
                 sgn=1, pre=None, rdma=None, lcopy=None),
            dict(d=1, ph=0, sbuf=sb2, rbufs=rb2, lbuf=lb2, ssem=ss2,
                 rsem=rs2, lsem=ls2, cred=cr2, tgt=left, crd_tgt=right,
                 sgn=-1, pre=None, rdma=None, lcopy=None),
            dict(d=1, ph=1, sbuf=sb3, rbufs=rb3, lbuf=lb3, ssem=ss3,
                 rsem=rs3, lsem=ls3, cred=cr3, tgt=left, crd_tgt=right,
                 sgn=-1, pre=None, rdma=None, lcopy=None),
        ]

        def group(lane, rl):
            return lane["d"] * (G // 2) + lane["ph"] + 2 * rl

        def cols(lane, rl):
            return pl.ds(group(lane, rl) * NW, NW)

        def signal_credit(lane):
            pl.semaphore_signal(lane["cred"], inc=1,
                                device_id=(lane["crd_tgt"],),
                                device_id_type=pl.DeviceIdType.MESH)

        def preload(lane, rl):
            lane["pre"] = pltpu.make_async_copy(
                p_hbm.at[pl.ds(my * MC, MC), cols(lane, rl)],
                lane["sbuf"], lane["lsem"])
            lane["pre"].start()

        def issue(lane, ls):
            rl, h = divmod(ls, N_HOPS)
            slot = ls % 2
            if ls >= 2:
                pl.semaphore_wait(lane["cred"], 1)
            if h == 0:
                lane["pre"].wait()
            src = (lane["sbuf"] if h <= 3
                   else lane["rbufs"].at[(ls - 1) % 2])
            lane["rdma"] = pltpu.make_async_remote_copy(
                src_ref=src,
                dst_ref=lane["rbufs"].at[slot],
                send_sem=lane["ssem"].at[slot],
                recv_sem=lane["rsem"].at[slot],
                device_id=(lane["tgt"],),
                device_id_type=pl.DeviceIdType.MESH,
            )
            lane["rdma"].start()
            if h < N_DEV - 1:
                c_next = mod4(my - lane["sgn"] * (h + 1))
                lane["lcopy"] = pltpu.make_async_copy(
                    p_hbm.at[pl.ds(c_next * MC, MC), cols(lane, rl)],
                    lane["lbuf"], lane["lsem"])
                lane["lcopy"].start()

        def finish(lane, ls):
            rl, h = divmod(ls, N_HOPS)
            slot = ls % 2
            D = dirstate[lane["d"]]
            lane["rdma"].wait()
            if h < N_DEV - 1:
                lane["lcopy"].wait()
            if h < N_DEV - 2:
                acc = (lane["rbufs"][slot].astype(jnp.float32)
                       + lane["lbuf"][...].astype(jnp.float32))
                lane["sbuf"][...] = acc.astype(jnp.bfloat16)
                signal_credit(lane)
            elif h == N_DEV - 2:
                acc = (lane["rbufs"][slot].astype(jnp.float32)
                       + lane["lbuf"][...].astype(jnp.float32))
                y = acc * jax.nn.sigmoid(acc)
                if D["out_copy"] is not None:
                    D["out_copy"].wait()
                D["fbuf"][...] = y
                lane["sbuf"][...] = y.astype(jnp.bfloat16)
                own = mod4(my + lane["sgn"])
                D["out_copy"] = pltpu.make_async_copy(
                    D["fbuf"],
                    out_hbm.at[pl.ds(own * MC, MC), cols(lane, rl)],
                    D["osem"])
                D["out_copy"].start()
                signal_credit(lane)
            else:
                a = h - (N_DEV - 1)
                c = mod4(my - lane["sgn"] * a)
                D["out_copy"].wait()
                D["fbuf"][...] = lane["rbufs"][slot].astype(jnp.float32)
                D["out_copy"] = pltpu.make_async_copy(
                    D["fbuf"],
                    out_hbm.at[pl.ds(c * MC, MC), cols(lane, rl)],
                    D["osem"])
                D["out_copy"].start()
                if h == 3 and rl + 1 < R_LANE:
                    preload(lane, rl + 1)
                if h == 4:
                    signal_credit(lane)
                if h == 5 and rl + 1 < R_LANE:
                    signal_credit(lane)
                    signal_credit(lane)

        for lane in lanes:
            preload(lane, 0)

        barrier = pltpu.get_barrier_semaphore()
        pl.semaphore_signal(barrier, inc=1, device_id=(left,),
                            device_id_type=pl.DeviceIdType.MESH)
        pl.semaphore_signal(barrier, inc=1, device_id=(right,),
                            device_id_type=pl.DeviceIdType.MESH)
        pl.semaphore_wait(barrier, 2)

        for lane in lanes:
            issue(lane, 0)
        for ls in range(1, LS_TOT):
            for lane in lanes:
                finish(lane, ls - 1)
                issue(lane, ls)
        for lane in lanes:
            finish(lane, LS_TOT - 1)
        for D in dirstate:
            D["out_copy"].wait()

    return pl.pallas_call(
        body,
        out_shape=jax.ShapeDtypeStruct((M, N), jnp.float32),
        in_specs=[pl.BlockSpec(memory_space=pl.ANY)],
        out_specs=pl.BlockSpec(memory_space=pl.ANY),
        scratch_shapes=[
            pltpu.VMEM((MC, NW), jnp.bfloat16),
            pltpu.VMEM((MC, NW), jnp.bfloat16),
            pltpu.VMEM((MC, NW), jnp.bfloat16),
            pltpu.VMEM((MC, NW), jnp.bfloat16),
            pltpu.VMEM((2, MC, NW), jnp.bfloat16),
            pltpu.VMEM((2, MC, NW), jnp.bfloat16),
            pltpu.VMEM((2, MC, NW), jnp.bfloat16),
            pltpu.VMEM((2, MC, NW), jnp.bfloat16),
            pltpu.VMEM((MC, NW), jnp.bfloat16),
            pltpu.VMEM((MC, NW), jnp.bfloat16),
            pltpu.VMEM((MC, NW), jnp.bfloat16),
            pltpu.VMEM((MC, NW), jnp.bfloat16),
            pltpu.VMEM((MC, NW), jnp.float32),
            pltpu.VMEM((MC, NW), jnp.float32),
            pltpu.SemaphoreType.DMA((2,)),
            pltpu.SemaphoreType.DMA((2,)),
            pltpu.SemaphoreType.DMA((2,)),
            pltpu.SemaphoreType.DMA((2,)),
            pltpu.SemaphoreType.DMA((2,)),
            pltpu.SemaphoreType.DMA((2,)),
            pltpu.SemaphoreType.DMA((2,)),
            pltpu.SemaphoreType.DMA((2,)),
            pltpu.SemaphoreType.DMA,
            pltpu.SemaphoreType.DMA,
            pltpu.SemaphoreType.DMA,
            pltpu.SemaphoreType.DMA,
            pltpu.SemaphoreType.DMA,
            pltpu.SemaphoreType.DMA,
            pltpu.SemaphoreType.REGULAR,
            pltpu.SemaphoreType.REGULAR,
            pltpu.SemaphoreType.REGULAR,
            pltpu.SemaphoreType.REGULAR,
        ],
        compiler_params=pltpu.CompilerParams(
            collective_id=0, vmem_limit_bytes=64 * 1024 * 1024),
    )(partial)
